# Initial kernel scaffold; baseline (speedup 1.0000x reference)
import jax
import jax.numpy as jnp
from jax.experimental import pallas as pl


def kernel(points):
    raise NotImplementedError("write your pallas kernel here")


# speedup vs baseline: 1.4733x; 1.4733x over previous
import functools

import jax
import jax.numpy as jnp
from jax import lax
from jax.experimental import pallas as pl
from jax.experimental.pallas import tpu as pltpu
from jax.experimental.pallas import tpu_sc as plsc

N = 2_000_000
G = 256
GN = G * G * G
NT = 16
P = N // NT
CH = 5000
NCH = P // CH
NGRP = CH // 16 + 1
IDXN = NGRP * 16 + 48
ZB = 32768
NZ = GN // NT // ZB


def _occupancy_body(points_hbm, grid_hbm, pts0, pts1, idx0, idx1, ones_v,
                    zbuf, zsem, psem0, psem1, ssem0, ssem1):
    sid = lax.axis_index("s")
    lane = jnp.arange(16, dtype=jnp.int32)

    def _zfill(i, _):
        zbuf[pl.ds(i * 16, 16)] = jnp.zeros((16,), jnp.int32)
        return _

    lax.fori_loop(0, ZB // 16, _zfill, None)

    def _ofill(i, _):
        ones_v[pl.ds(i * 16, 16)] = jnp.ones((16,), jnp.int32)
        return _

    lax.fori_loop(0, IDXN // 16, _ofill, None)

    neg = jnp.full((16,), -1, jnp.int32)
    for t in range(NGRP * 16, IDXN, 16):
        idx0[pl.ds(t, 16)] = neg
        idx1[pl.ds(t, 16)] = neg

    zcopies = [
        pltpu.async_copy(
            zbuf, grid_hbm.at[pl.ds((sid * NZ + k) * ZB, ZB)], zsem)
        for k in range(NZ)
    ]

    base = sid * P
    pts_bufs = (pts0, pts1)
    idx_bufs = (idx0, idx1)
    psems = (psem0, psem1)
    ssems = (ssem0, ssem1)

    pload = [None] * NCH
    pload[0] = pltpu.async_copy(
        points_hbm.at[pl.ds(base * 3, CH * 3)], pts_bufs[0], psems[0])

    for c in zcopies:
        c.wait()
    plsc.subcore_barrier()

    def _compute(pts, idx):
        def body(g, _):
            r3 = (jnp.minimum(g * 16, CH - 16) + lane) * 3
            x = plsc.load_gather(pts, [r3])
            y = plsc.load_gather(pts, [r3 + 1])
            z = plsc.load_gather(pts, [r3 + 2])
            fx = x * 256.0
            fy = y * 256.0
            fz = z * 256.0
            inb = (fx <= 255.0) & (fy <= 255.0) & (fz <= 255.0)
            v = ((fx.astype(jnp.int32) << 16)
                 | (fy.astype(jnp.int32) << 8)
                 | fz.astype(jnp.int32))
            idx[pl.ds(g * 16, 16)] = jnp.where(inb, v, -1)
            return _

        lax.fori_loop(0, NGRP, body, None)

    scat = [None] * NCH
    for c in range(NCH):
        b = c % 2
        pload[c].wait()
        if c + 1 < NCH:
            pload[c + 1] = pltpu.async_copy(
                points_hbm.at[pl.ds((base + (c + 1) * CH) * 3, CH * 3)],
                pts_bufs[(c + 1) % 2], psems[(c + 1) % 2])
        if c >= 2:
            scat[c - 2].wait()
        _compute(pts_bufs[b], idx_bufs[b])
        scat[c] = pltpu.async_copy(
            ones_v,
            grid_hbm.at[plsc.Indices(idx_bufs[b], ignored_value=-1)],
            ssems[b])
    scat[NCH - 2].wait()
    scat[NCH - 1].wait()


@functools.partial(jax.jit, donate_argnums=())
def _occupancy(points):
    mesh = plsc.VectorSubcoreMesh(
        core_axis_name="c", subcore_axis_name="s", num_cores=1)
    run = pl.kernel(
        _occupancy_body,
        out_type=jax.ShapeDtypeStruct((GN,), jnp.int32),
        mesh=mesh,
        compiler_params=pltpu.CompilerParams(needs_layout_passes=False),
        scratch_types=[
            pltpu.VMEM((CH * 3,), jnp.float32),
            pltpu.VMEM((CH * 3,), jnp.float32),
            pltpu.VMEM((IDXN,), jnp.int32),
            pltpu.VMEM((IDXN,), jnp.int32),
            pltpu.VMEM((IDXN,), jnp.int32),
            pltpu.VMEM((ZB,), jnp.int32),
            pltpu.SemaphoreType.DMA,
            pltpu.SemaphoreType.DMA,
            pltpu.SemaphoreType.DMA,
            pltpu.SemaphoreType.DMA,
            pltpu.SemaphoreType.DMA,
        ],
    )
    return run(points.reshape(-1))


def kernel(points):
    grid32 = _occupancy(points)
    return grid32.reshape(G, G, G).astype(jnp.bool_)


# speedup vs baseline: 5.8424x; 3.9655x over previous
import functools

import jax
import jax.numpy as jnp
from jax import lax
from jax.experimental import pallas as pl
from jax.experimental.pallas import tpu as pltpu
from jax.experimental.pallas import tpu_sc as plsc

N = 2_000_000
G = 256
GN = G * G * G
NT = 16
P = N // NT
CH = 5000
NCH = P // CH
NGRP = CH // 16 + 1
IDXN = NGRP * 16 + 48
ZB = 32768
NZ = GN // NT // ZB


def _occupancy_body(x_hbm, y_hbm, z_hbm, grid_hbm,
                    xb0, yb0, zb0, xb1, yb1, zb1, idx0, idx1, ones_v,
                    zbuf, zsem, psem0, psem1, ssem0, ssem1):
    sid = lax.axis_index("s")

    def _zfill(i, _):
        zbuf[pl.ds(i * 16, 16)] = jnp.zeros((16,), jnp.int32)
        return _

    lax.fori_loop(0, ZB // 16, _zfill, None)

    def _ofill(i, _):
        ones_v[pl.ds(i * 16, 16)] = jnp.ones((16,), jnp.int32)
        return _

    lax.fori_loop(0, IDXN // 16, _ofill, None)

    neg = jnp.full((16,), -1, jnp.int32)
    for t in range(NGRP * 16, IDXN, 16):
        idx0[pl.ds(t, 16)] = neg
        idx1[pl.ds(t, 16)] = neg

    zcopies = [
        pltpu.async_copy(
            zbuf, grid_hbm.at[pl.ds((sid * NZ + k) * ZB, ZB)], zsem)
        for k in range(NZ)
    ]

    base = sid * P
    pts_bufs = ((xb0, yb0, zb0), (xb1, yb1, zb1))
    idx_bufs = (idx0, idx1)
    psems = (psem0, psem1)
    ssems = (ssem0, ssem1)

    def _start_load(c):
        b = pts_bufs[c % 2]
        sem = psems[c % 2]
        sl = pl.ds(base + c * CH, CH)
        return (pltpu.async_copy(x_hbm.at[sl], b[0], sem),
                pltpu.async_copy(y_hbm.at[sl], b[1], sem),
                pltpu.async_copy(z_hbm.at[sl], b[2], sem))

    pload = [None] * NCH
    pload[0] = _start_load(0)

    for c in zcopies:
        c.wait()
    plsc.subcore_barrier()

    def _compute(bufs, idx):
        xb, yb, zb = bufs

        def body(g, _):
            row = pl.ds(jnp.minimum(g * 16, CH - 16), 16)
            fx = xb[row] * 256.0
            fy = yb[row] * 256.0
            fz = zb[row] * 256.0
            inb = (fx <= 255.0) & (fy <= 255.0) & (fz <= 255.0)
            v = ((fx.astype(jnp.int32) << 16)
                 | (fy.astype(jnp.int32) << 8)
                 | fz.astype(jnp.int32))
            idx[pl.ds(g * 16, 16)] = jnp.where(inb, v, -1)
            return _

        lax.fori_loop(0, NGRP, body, None)

    scat = [None] * NCH
    for c in range(NCH):
        b = c % 2
        for cp in pload[c]:
            cp.wait()
        if c + 1 < NCH:
            pload[c + 1] = _start_load(c + 1)
        if c >= 2:
            scat[c - 2].wait()
        _compute(pts_bufs[b], idx_bufs[b])
        scat[c] = pltpu.async_copy(
            ones_v,
            grid_hbm.at[plsc.Indices(idx_bufs[b], ignored_value=-1)],
            ssems[b])
    scat[NCH - 2].wait()
    scat[NCH - 1].wait()


@jax.jit
def _occupancy(points):
    mesh = plsc.VectorSubcoreMesh(
        core_axis_name="c", subcore_axis_name="s", num_cores=1)
    run = pl.kernel(
        _occupancy_body,
        out_type=jax.ShapeDtypeStruct((GN,), jnp.int32),
        mesh=mesh,
        compiler_params=pltpu.CompilerParams(needs_layout_passes=False),
        scratch_types=[
            pltpu.VMEM((CH,), jnp.float32),
            pltpu.VMEM((CH,), jnp.float32),
            pltpu.VMEM((CH,), jnp.float32),
            pltpu.VMEM((CH,), jnp.float32),
            pltpu.VMEM((CH,), jnp.float32),
            pltpu.VMEM((CH,), jnp.float32),
            pltpu.VMEM((IDXN,), jnp.int32),
            pltpu.VMEM((IDXN,), jnp.int32),
            pltpu.VMEM((IDXN,), jnp.int32),
            pltpu.VMEM((ZB,), jnp.int32),
            pltpu.SemaphoreType.DMA,
            pltpu.SemaphoreType.DMA,
            pltpu.SemaphoreType.DMA,
            pltpu.SemaphoreType.DMA,
            pltpu.SemaphoreType.DMA,
        ],
    )
    grid32 = run(points[:, 0], points[:, 1], points[:, 2])
    return grid32.astype(jnp.bool_).reshape(G, G, G)


def kernel(points):
    return _occupancy(points)
